# baseline (speedup 1.0000x reference)
import functools

import jax
import jax.numpy as jnp
from jax import lax
from jax.experimental import pallas as pl
from jax.experimental.pallas import tpu as pltpu
from jax.experimental.pallas import tpu_sc as plsc

NUM_DATA = 100000
NUM_CLASSES = 128
BATCH = 16384

NC = 2
NS = 16
NW = NC * NS
B_PER_W = BATCH // NW
IDX_CHUNK = 64
N_CHUNKS = B_PER_W // IDX_CHUNK
LANES = 16


def _gather_sigmoid_kernel(table_hbm, idx_hbm, out_hbm, idx_v, rows_v, gsem, ssem):
    wid = lax.axis_index("s") * NC + lax.axis_index("c")
    base = wid * B_PER_W

    pltpu.sync_copy(idx_hbm.at[wid, pl.ds(0, 1)], idx_v.at[pl.ds(0, 1)])

    gathers = [
        pltpu.async_copy(
            table_hbm.at[idx_v.at[0]],
            rows_v.at[pl.ds(0, IDX_CHUNK)],
            gsem.at[0],
        )
    ]
    pltpu.sync_copy(idx_hbm.at[wid, pl.ds(1, N_CHUNKS - 1)],
                    idx_v.at[pl.ds(1, N_CHUNKS - 1)])
    for j in range(1, N_CHUNKS):
        gathers.append(
            pltpu.async_copy(
                table_hbm.at[idx_v.at[j]],
                rows_v.at[pl.ds(j * IDX_CHUNK, IDX_CHUNK)],
                gsem.at[j],
            )
        )

    groups = [(0, 1), (1, 3), (3, 5), (5, 7), (7, 8)]
    stores = []
    for k, (g0, g1) in enumerate(groups):
        for g in range(g0, g1):
            gathers[g].wait()
        lo = g0 * IDX_CHUNK
        hi = g1 * IDX_CHUNK

        @plsc.parallel_loop(lo, hi, 1, unroll=2)
        def row_body(r):
            for c in range(NUM_CLASSES // LANES):
                x = rows_v[r, pl.ds(c * LANES, LANES)]
                rows_v[r, pl.ds(c * LANES, LANES)] = 1.0 / (1.0 + jnp.exp(-x))

        stores.append(
            pltpu.async_copy(
                rows_v.at[pl.ds(lo, hi - lo)],
                out_hbm.at[pl.ds(base + lo, hi - lo)],
                ssem.at[k],
            )
        )
    for s in stores:
        s.wait()


@functools.partial(jax.jit, static_argnums=())
def _run(table, idx):
    mesh = plsc.VectorSubcoreMesh(core_axis_name="c", subcore_axis_name="s")
    return pl.kernel(
        _gather_sigmoid_kernel,
        mesh=mesh,
        out_type=jax.ShapeDtypeStruct((BATCH, NUM_CLASSES), jnp.float32),
        scratch_types=[
            pltpu.VMEM((N_CHUNKS, IDX_CHUNK), jnp.int32),
            pltpu.VMEM((B_PER_W, NUM_CLASSES), jnp.float32),
            pltpu.SemaphoreType.DMA((N_CHUNKS,)),
            pltpu.SemaphoreType.DMA((5,)),
        ],
    )(table, idx)


def kernel(logits, indices):
    idx = indices.astype(jnp.int32).reshape(NW, N_CHUNKS, IDX_CHUNK)
    return _run(logits, idx)


# speedup vs baseline: 1.0159x; 1.0159x over previous
import functools

import jax
import jax.numpy as jnp
from jax import lax
from jax.experimental import pallas as pl
from jax.experimental.pallas import tpu as pltpu
from jax.experimental.pallas import tpu_sc as plsc

NUM_DATA = 100000
NUM_CLASSES = 128
BATCH = 16384

NC = 2
NS = 16
NW = NC * NS
B_PER_W = BATCH // NW
IDX_CHUNK = 64
N_CHUNKS = B_PER_W // IDX_CHUNK
LANES = 16


def _gather_sigmoid_kernel(table_hbm, idx_hbm, out_hbm, idx_v, rows_v, gsem, ssem):
    wid = lax.axis_index("s") * NC + lax.axis_index("c")
    base = wid * B_PER_W

    pltpu.sync_copy(idx_hbm.at[wid], idx_v)

    gathers = []
    for j in range(N_CHUNKS):
        gathers.append(
            pltpu.async_copy(
                table_hbm.at[idx_v.at[j]],
                rows_v.at[pl.ds(j * IDX_CHUNK, IDX_CHUNK)],
                gsem.at[j],
            )
        )

    groups = [(0, 1), (1, 3), (3, 5), (5, 7), (7, 8)]
    stores = []
    for k, (g0, g1) in enumerate(groups):
        for g in range(g0, g1):
            gathers[g].wait()
        lo = g0 * IDX_CHUNK
        hi = g1 * IDX_CHUNK

        @plsc.parallel_loop(lo, hi, 1, unroll=2)
        def row_body(r):
            for c in range(NUM_CLASSES // LANES):
                x = rows_v[r, pl.ds(c * LANES, LANES)]
                rows_v[r, pl.ds(c * LANES, LANES)] = 1.0 / (1.0 + jnp.exp(-x))

        stores.append(
            pltpu.async_copy(
                rows_v.at[pl.ds(lo, hi - lo)],
                out_hbm.at[pl.ds(base + lo, hi - lo)],
                ssem.at[k],
            )
        )
    for s in stores:
        s.wait()


@functools.partial(jax.jit, static_argnums=())
def _run(table, idx):
    mesh = plsc.VectorSubcoreMesh(core_axis_name="c", subcore_axis_name="s")
    return pl.kernel(
        _gather_sigmoid_kernel,
        mesh=mesh,
        out_type=jax.ShapeDtypeStruct((BATCH, NUM_CLASSES), jnp.float32),
        scratch_types=[
            pltpu.VMEM((N_CHUNKS, IDX_CHUNK), jnp.int32),
            pltpu.VMEM((B_PER_W, NUM_CLASSES), jnp.float32),
            pltpu.SemaphoreType.DMA((N_CHUNKS,)),
            pltpu.SemaphoreType.DMA((5,)),
        ],
    )(table, idx)


def kernel(logits, indices):
    idx = indices.astype(jnp.int32).reshape(NW, N_CHUNKS, IDX_CHUNK)
    return _run(logits, idx)
